# baseline (speedup 1.0000x reference)
import functools

import jax
import jax.numpy as jnp
from jax import lax
from jax.experimental import pallas as pl
from jax.experimental.pallas import tpu as pltpu
from jax.experimental.pallas import tpu_sc as plsc

_NC = 2
_NS = 16
_NW = _NC * _NS
_L = 16


def _gather_kernel(npad_per_batch, hw, c, rows_per_worker):
  mesh = plsc.VectorSubcoreMesh(
      core_axis_name="c", subcore_axis_name="s", num_cores=_NC,
      num_subcores=_NS)
  total_rows = _NW * rows_per_worker
  workers_per_batch = npad_per_batch // rows_per_worker

  @functools.partial(
      pl.kernel,
      mesh=mesh,
      out_type=jax.ShapeDtypeStruct((total_rows, c), jnp.float32),
      scratch_types=[
          pltpu.VMEM((rows_per_worker,), jnp.int32),
          pltpu.VMEM((rows_per_worker, c), jnp.float32),
          pltpu.SemaphoreType.DMA,
      ],
      compiler_params=pltpu.CompilerParams(use_tc_tiling_on_sc=False),
  )
  def k(flat_hbm, idx_hbm, out_hbm, idx_v, rows_v, sem):
    wid = lax.axis_index("s") * _NC + lax.axis_index("c")
    base = wid * rows_per_worker
    batch = wid // workers_per_batch
    off = batch * hw
    pltpu.sync_copy(idx_hbm.at[pl.ds(base, rows_per_worker)], idx_v)
    for j in range(rows_per_worker // _L):
      sl = pl.ds(j * _L, _L)
      idx_v[sl] = idx_v[sl] + off
    pltpu.async_copy(flat_hbm.at[idx_v], rows_v, sem).wait()
    pltpu.sync_copy(rows_v, out_hbm.at[pl.ds(base, rows_per_worker)])

  return k


def kernel(feat, index):
  b, h, w, c = feat.shape
  hw = h * w
  n = index.shape[1]
  npad = n
  while (b * npad) % (8 * _NW) != 0:
    npad += 1
  rows_per_worker = b * npad // _NW
  idx_pad = jnp.pad(index, ((0, 0), (0, npad - n)))
  flat = feat.reshape(b * hw, c)
  k = _gather_kernel(npad, hw, c, rows_per_worker)
  out = k(flat, idx_pad.reshape(b * npad))
  return out.reshape(b, npad, c)[:, :n, :]


# speedup vs baseline: 2.3646x; 2.3646x over previous
import functools

import jax
import jax.numpy as jnp
from jax import lax
from jax.experimental import pallas as pl
from jax.experimental.pallas import tpu as pltpu
from jax.experimental.pallas import tpu_sc as plsc

_NC = 2
_NS = 16
_NW = _NC * _NS
_L = 16


def _gather_kernel(npad_per_batch, hw, c, rows_per_worker):
  mesh = plsc.VectorSubcoreMesh(
      core_axis_name="c", subcore_axis_name="s", num_cores=_NC,
      num_subcores=_NS)
  total_rows = _NW * rows_per_worker
  workers_per_batch = npad_per_batch // rows_per_worker

  @functools.partial(
      pl.kernel,
      mesh=mesh,
      out_type=jax.ShapeDtypeStruct((total_rows, c), jnp.float32),
      scratch_types=[
          pltpu.VMEM((rows_per_worker,), jnp.int32),
          pltpu.VMEM((rows_per_worker, c), jnp.float32),
          pltpu.SemaphoreType.DMA,
      ],
  )
  def k(flat_hbm, idx_hbm, out_hbm, idx_v, rows_v, sem):
    wid = lax.axis_index("s") * _NC + lax.axis_index("c")
    base = wid * rows_per_worker
    batch = wid // workers_per_batch
    off = batch * hw
    pltpu.sync_copy(idx_hbm.at[pl.ds(base, rows_per_worker)], idx_v)
    copies = []
    for jc in range(rows_per_worker // _L):
      v = idx_v[pl.ds(jc * _L, _L)] + off
      for e in range(_L):
        j = jc * _L + e
        copies.append(
            pltpu.make_async_copy(
                flat_hbm.at[pl.ds(v[e], 1), :], rows_v.at[pl.ds(j, 1), :],
                sem))
    for cp in copies:
      cp.start()
    for cp in copies:
      cp.wait()
    pltpu.sync_copy(rows_v, out_hbm.at[pl.ds(base, rows_per_worker)])

  return k


def kernel(feat, index):
  b, h, w, c = feat.shape
  hw = h * w
  n = index.shape[1]
  npad = n
  while (b * npad) % (8 * _NW) != 0:
    npad += 1
  rows_per_worker = b * npad // _NW
  idx_pad = jnp.pad(index, ((0, 0), (0, npad - n)))
  flat = feat.reshape(b * hw, c)
  k = _gather_kernel(npad, hw, c, rows_per_worker)
  out = k(flat, idx_pad.reshape(b * npad))
  return out.reshape(b, npad, c)[:, :n, :]


# speedup vs baseline: 4.5085x; 1.9067x over previous
import functools

import jax
import jax.numpy as jnp
from jax import lax
from jax.experimental import pallas as pl
from jax.experimental.pallas import tpu as pltpu
from jax.experimental.pallas import tpu_sc as plsc

_NC = 2
_NS = 16
_NW = _NC * _NS
_L = 16
_RING = 4


def _gather_kernel(h, w, c, rows_per_worker):
  mesh = plsc.VectorSubcoreMesh(
      core_axis_name="c", subcore_axis_name="s", num_cores=_NC,
      num_subcores=_NS)
  total_rows = _NW * rows_per_worker
  n_full = w // 128

  @functools.partial(
      pl.kernel,
      mesh=mesh,
      out_type=jax.ShapeDtypeStruct((total_rows, c), jnp.float32),
      scratch_types=[
          pltpu.VMEM((rows_per_worker,), jnp.int32),
          pltpu.VMEM((rows_per_worker,), jnp.int32),
          pltpu.VMEM((rows_per_worker * _L,), jnp.int32),
          pltpu.VMEM((rows_per_worker, c), jnp.float32),
          pltpu.VMEM((_RING, c, w), jnp.float32),
          [pltpu.SemaphoreType.DMA] * _RING,
      ],
      compiler_params=pltpu.CompilerParams(needs_layout_passes=False),
  )
  def k(tcw_hbm, rb_hbm, wt_hbm, lane_hbm, out_hbm, rb_v, wt_v, lane_v,
        rows_v, slabs_v, sems):
    wid = lax.axis_index("s") * _NC + lax.axis_index("c")
    base = wid * rows_per_worker
    pltpu.sync_copy(rb_hbm.at[pl.ds(base, rows_per_worker)], rb_v)
    pltpu.sync_copy(wt_hbm.at[pl.ds(base, rows_per_worker)], wt_v)
    pltpu.sync_copy(
        lane_hbm.at[pl.ds(base * _L, rows_per_worker * _L)], lane_v)
    chunks = []
    for jc in range(rows_per_worker // _L):
      sl = pl.ds(jc * _L, _L)
      chunks.append((rb_v[sl], wt_v[sl]))

    def fields(j):
      rb, wt = chunks[j // _L]
      e = j % _L
      return pl.multiple_of(rb[e], c), wt[e]

    def start(j):
      rbe, wte = fields(j)
      slot = j % _RING

      @pl.when(wte < n_full)
      def _():
        off = pl.multiple_of(wte * 128, 128)
        pltpu.make_async_copy(
            tcw_hbm.at[pl.ds(rbe, c), pl.ds(off, 128)],
            slabs_v.at[slot, :, pl.ds(0, 128)], sems[slot]).start()

      @pl.when(wte == n_full)
      def _():
        pltpu.make_async_copy(
            tcw_hbm.at[pl.ds(rbe, c), :],
            slabs_v.at[slot], sems[slot]).start()

    def finish(j):
      _, wte = fields(j)
      slot = j % _RING

      @pl.when(wte < n_full)
      def _():
        pltpu.make_async_copy(
            tcw_hbm.at[pl.ds(0, c), pl.ds(0, 128)],
            slabs_v.at[slot, :, pl.ds(0, 128)], sems[slot]).wait()

      @pl.when(wte == n_full)
      def _():
        pltpu.make_async_copy(
            tcw_hbm.at[pl.ds(0, c), :],
            slabs_v.at[slot], sems[slot]).wait()

      lane_s = lane_v[pl.ds(j * _L, _L)]
      for kk in range(c // _L):
        ci = lax.iota(jnp.int32, _L) + kk * _L
        val = plsc.load_gather(slabs_v.at[slot], [ci, lane_s])
        rows_v[j, pl.ds(kk * _L, _L)] = val

    for j in range(_RING):
      start(j)
    for j in range(rows_per_worker):
      finish(j)
      if j + _RING < rows_per_worker:
        start(j + _RING)
    pltpu.sync_copy(rows_v, out_hbm.at[pl.ds(base, rows_per_worker)])

  return k


def kernel(feat, index):
  b, h, w, c = feat.shape
  n = index.shape[1]
  npad = n
  while (b * npad) % (8 * _NW) != 0:
    npad += 1
  rows_per_worker = b * npad // _NW
  idx_pad = jnp.pad(index, ((0, 0), (0, npad - n))).reshape(b * npad)
  n_full = w // 128
  batch_of = jnp.arange(b * npad, dtype=jnp.int32) // npad
  hh = idx_pad // w
  cw = idx_pad - hh * w
  rb = ((batch_of * h + hh) * c).astype(jnp.int32)
  wt = cw // 128
  lane = jnp.where(wt == n_full, cw, cw - wt * 128).astype(jnp.int32)
  lane_b = jnp.broadcast_to(
      lane[:, None], (lane.shape[0], _L)).reshape(-1)
  tcw = jnp.transpose(feat, (0, 1, 3, 2)).reshape(b * h * c, w)
  k = _gather_kernel(h, w, c, rows_per_worker)
  out = k(tcw, rb, wt.astype(jnp.int32), lane_b)
  return out.reshape(b, npad, c)[:, :n, :]


# speedup vs baseline: 4.5584x; 1.0111x over previous
import functools

import jax
import jax.numpy as jnp
from jax import lax
from jax.experimental import pallas as pl
from jax.experimental.pallas import tpu as pltpu
from jax.experimental.pallas import tpu_sc as plsc

_NC = 2
_NS = 16
_NW = _NC * _NS
_L = 16
_RING = 4


def _gather_kernel(h, w, c, rows_per_worker):
  mesh = plsc.VectorSubcoreMesh(
      core_axis_name="c", subcore_axis_name="s", num_cores=_NC,
      num_subcores=_NS)
  total_rows = _NW * rows_per_worker
  n_full = w // 128

  @functools.partial(
      pl.kernel,
      mesh=mesh,
      out_type=jax.ShapeDtypeStruct((total_rows, c), jnp.float32),
      scratch_types=[
          pltpu.VMEM((rows_per_worker,), jnp.int32),
          pltpu.VMEM((rows_per_worker,), jnp.int32),
          pltpu.VMEM((rows_per_worker * _L,), jnp.int32),
          pltpu.VMEM((rows_per_worker, c), jnp.float32),
          pltpu.VMEM((_RING, c, w), jnp.float32),
          [pltpu.SemaphoreType.DMA] * _RING,
      ],
      compiler_params=pltpu.CompilerParams(needs_layout_passes=False),
  )
  def k(tcw_hbm, rb_hbm, wt_hbm, lane_hbm, out_hbm, rb_v, wt_v, lane_v,
        rows_v, slabs_v, sems):
    wid = lax.axis_index("s") * _NC + lax.axis_index("c")
    base = wid * rows_per_worker
    pltpu.sync_copy(rb_hbm.at[pl.ds(base, rows_per_worker)], rb_v)
    pltpu.sync_copy(wt_hbm.at[pl.ds(base, rows_per_worker)], wt_v)
    chunks = []
    for jc in range(rows_per_worker // _L):
      sl = pl.ds(jc * _L, _L)
      chunks.append((rb_v[sl], wt_v[sl]))

    def fields(j):
      rb, wt = chunks[j // _L]
      e = j % _L
      return pl.multiple_of(rb[e], c), wt[e]

    def start(j):
      rbe, wte = fields(j)
      slot = j % _RING

      @pl.when(wte < n_full)
      def _():
        off = pl.multiple_of(wte * 128, 128)
        pltpu.make_async_copy(
            tcw_hbm.at[pl.ds(rbe, c), pl.ds(off, 128)],
            slabs_v.at[slot, :, pl.ds(0, 128)], sems[slot]).start()

      @pl.when(wte == n_full)
      def _():
        pltpu.make_async_copy(
            tcw_hbm.at[pl.ds(rbe, c), :],
            slabs_v.at[slot], sems[slot]).start()

    def finish(j):
      _, wte = fields(j)
      slot = j % _RING

      @pl.when(wte < n_full)
      def _():
        pltpu.make_async_copy(
            tcw_hbm.at[pl.ds(0, c), pl.ds(0, 128)],
            slabs_v.at[slot, :, pl.ds(0, 128)], sems[slot]).wait()

      @pl.when(wte == n_full)
      def _():
        pltpu.make_async_copy(
            tcw_hbm.at[pl.ds(0, c), :],
            slabs_v.at[slot], sems[slot]).wait()

      lane_s = lane_v[pl.ds(j * _L, _L)]
      for kk in range(c // _L):
        ci = lax.iota(jnp.int32, _L) + kk * _L
        val = plsc.load_gather(slabs_v.at[slot], [ci, lane_s])
        rows_v[j, pl.ds(kk * _L, _L)] = val

    for j in range(_RING):
      start(j)
    pltpu.sync_copy(
        lane_hbm.at[pl.ds(base * _L, rows_per_worker * _L)], lane_v)
    for j in range(rows_per_worker):
      finish(j)
      if j + _RING < rows_per_worker:
        start(j + _RING)
    pltpu.sync_copy(rows_v, out_hbm.at[pl.ds(base, rows_per_worker)])

  return k


def kernel(feat, index):
  b, h, w, c = feat.shape
  n = index.shape[1]
  npad = n
  while (b * npad) % (8 * _NW) != 0:
    npad += 1
  rows_per_worker = b * npad // _NW
  idx_pad = jnp.pad(index, ((0, 0), (0, npad - n))).reshape(b * npad)
  n_full = w // 128
  batch_of = jnp.arange(b * npad, dtype=jnp.int32) // npad
  hh = idx_pad // w
  cw = idx_pad - hh * w
  rb = ((batch_of * h + hh) * c).astype(jnp.int32)
  wt = cw // 128
  lane = jnp.where(wt == n_full, cw, cw - wt * 128).astype(jnp.int32)
  lane_b = jnp.broadcast_to(
      lane[:, None], (lane.shape[0], _L)).reshape(-1)
  tcw = jnp.transpose(feat, (0, 1, 3, 2)).reshape(b * h * c, w)
  k = _gather_kernel(h, w, c, rows_per_worker)
  out = k(tcw, rb, wt.astype(jnp.int32), lane_b)
  return out.reshape(b, npad, c)[:, :n, :]


# speedup vs baseline: 6.1825x; 1.3563x over previous
import functools

import jax
import jax.numpy as jnp
from jax import lax
from jax.experimental import pallas as pl
from jax.experimental.pallas import tpu as pltpu
from jax.experimental.pallas import tpu_sc as plsc

_NC = 2
_NS = 16
_NW = _NC * _NS
_L = 16
_RING = 4


def _gather_kernel(h, w, c, rows_per_worker):
  mesh = plsc.VectorSubcoreMesh(
      core_axis_name="c", subcore_axis_name="s", num_cores=_NC,
      num_subcores=_NS)
  total_rows = _NW * rows_per_worker
  n_full = w // 128

  @functools.partial(
      pl.kernel,
      mesh=mesh,
      out_type=jax.ShapeDtypeStruct((total_rows, c), jnp.float32),
      scratch_types=[
          pltpu.VMEM((rows_per_worker,), jnp.int32),
          pltpu.VMEM((rows_per_worker,), jnp.int32),
          pltpu.VMEM((rows_per_worker * _L,), jnp.int32),
          pltpu.VMEM((rows_per_worker, c), jnp.float32),
          pltpu.VMEM((_RING, c, 128), jnp.float32),
          [pltpu.SemaphoreType.DMA] * _RING,
      ],
      compiler_params=pltpu.CompilerParams(needs_layout_passes=False),
  )
  def k(tcw_hbm, rb_hbm, wt_hbm, lane_hbm, out_hbm, rb_v, wt_v, lane_v,
        rows_v, slabs_v, sems):
    wid = lax.axis_index("s") * _NC + lax.axis_index("c")
    base = wid * rows_per_worker
    pltpu.sync_copy(rb_hbm.at[pl.ds(base, rows_per_worker)], rb_v)
    pltpu.sync_copy(wt_hbm.at[pl.ds(base, rows_per_worker)], wt_v)
    chunks = []
    for jc in range(rows_per_worker // _L):
      sl = pl.ds(jc * _L, _L)
      chunks.append((rb_v[sl], wt_v[sl]))

    def fields(j):
      rb, wt = chunks[j // _L]
      e = j % _L
      return pl.multiple_of(rb[e], c), wt[e]

    def start(j):
      rbe, wte = fields(j)
      slot = j % _RING
      off = pl.multiple_of(wte * 128, 128)
      pltpu.make_async_copy(
          tcw_hbm.at[pl.ds(rbe, c), pl.ds(off, 128)],
          slabs_v.at[slot], sems[slot]).start()

    def finish(j):
      slot = j % _RING
      pltpu.make_async_copy(
          tcw_hbm.at[pl.ds(0, c), pl.ds(0, 128)],
          slabs_v.at[slot], sems[slot]).wait()

      lane_s = lane_v[pl.ds(j * _L, _L)]
      for kk in range(c // _L):
        ci = lax.iota(jnp.int32, _L) + kk * _L
        val = plsc.load_gather(slabs_v.at[slot], [ci, lane_s])
        rows_v[j, pl.ds(kk * _L, _L)] = val

    for j in range(_RING):
      start(j)
    pltpu.sync_copy(
        lane_hbm.at[pl.ds(base * _L, rows_per_worker * _L)], lane_v)
    for j in range(rows_per_worker):
      finish(j)
      if j + _RING < rows_per_worker:
        start(j + _RING)
    pltpu.sync_copy(rows_v, out_hbm.at[pl.ds(base, rows_per_worker)])

  return k


def kernel(feat, index):
  b, h, w, c = feat.shape
  n = index.shape[1]
  npad = n
  while (b * npad) % (8 * _NW) != 0:
    npad += 1
  rows_per_worker = b * npad // _NW
  idx_pad = jnp.pad(index, ((0, 0), (0, npad - n))).reshape(b * npad)
  n_full = w // 128
  batch_of = jnp.arange(b * npad, dtype=jnp.int32) // npad
  hh = idx_pad // w
  cw = idx_pad - hh * w
  rb = ((batch_of * h + hh) * c).astype(jnp.int32)
  wt = cw // 128
  lane = (cw - wt * 128).astype(jnp.int32)
  lane_b = jnp.broadcast_to(
      lane[:, None], (lane.shape[0], _L)).reshape(-1)
  tcw = jnp.transpose(feat, (0, 1, 3, 2)).reshape(b * h * c, w)
  k = _gather_kernel(h, w, c, rows_per_worker)
  out = k(tcw, rb, wt.astype(jnp.int32), lane_b)
  return out.reshape(b, npad, c)[:, :n, :]


# speedup vs baseline: 6.8221x; 1.1034x over previous
import functools

import jax
import jax.numpy as jnp
from jax import lax
from jax.experimental import pallas as pl
from jax.experimental.pallas import tpu as pltpu
from jax.experimental.pallas import tpu_sc as plsc

_NC = 2
_NS = 16
_NW = _NC * _NS
_L = 16
_RING = 8


def _gather_kernel(h, w, c, rows_per_worker):
  mesh = plsc.VectorSubcoreMesh(
      core_axis_name="c", subcore_axis_name="s", num_cores=_NC,
      num_subcores=_NS)
  total_rows = _NW * rows_per_worker
  n_full = w // 128

  @functools.partial(
      pl.kernel,
      mesh=mesh,
      out_type=jax.ShapeDtypeStruct((total_rows, c), jnp.float32),
      scratch_types=[
          pltpu.VMEM((rows_per_worker,), jnp.int32),
          pltpu.VMEM((rows_per_worker,), jnp.int32),
          pltpu.VMEM((rows_per_worker * _L,), jnp.int32),
          pltpu.VMEM((rows_per_worker, c), jnp.float32),
          pltpu.VMEM((_RING, c, 128), jnp.float32),
          [pltpu.SemaphoreType.DMA] * _RING,
      ],
      compiler_params=pltpu.CompilerParams(needs_layout_passes=False),
  )
  def k(tcw_hbm, rb_hbm, wt_hbm, lane_hbm, out_hbm, rb_v, wt_v, lane_v,
        rows_v, slabs_v, sems):
    wid = lax.axis_index("s") * _NC + lax.axis_index("c")
    base = wid * rows_per_worker
    pltpu.sync_copy(rb_hbm.at[pl.ds(base, rows_per_worker)], rb_v)
    pltpu.sync_copy(wt_hbm.at[pl.ds(base, rows_per_worker)], wt_v)
    chunks = []
    for jc in range(rows_per_worker // _L):
      sl = pl.ds(jc * _L, _L)
      chunks.append((rb_v[sl], wt_v[sl]))

    def fields(j):
      rb, wt = chunks[j // _L]
      e = j % _L
      return pl.multiple_of(rb[e], c), wt[e]

    def start(j):
      rbe, wte = fields(j)
      slot = j % _RING
      off = pl.multiple_of(wte * 128, 128)
      pltpu.make_async_copy(
          tcw_hbm.at[pl.ds(rbe, c), pl.ds(off, 128)],
          slabs_v.at[slot], sems[slot]).start()

    def finish(j):
      slot = j % _RING
      pltpu.make_async_copy(
          tcw_hbm.at[pl.ds(0, c), pl.ds(0, 128)],
          slabs_v.at[slot], sems[slot]).wait()

      lane_s = lane_v[pl.ds(j * _L, _L)]
      for kk in range(c // _L):
        ci = lax.iota(jnp.int32, _L) + kk * _L
        val = plsc.load_gather(slabs_v.at[slot], [ci, lane_s])
        rows_v[j, pl.ds(kk * _L, _L)] = val

    for j in range(_RING):
      start(j)
    pltpu.sync_copy(
        lane_hbm.at[pl.ds(base * _L, rows_per_worker * _L)], lane_v)
    for j in range(rows_per_worker):
      finish(j)
      if j + _RING < rows_per_worker:
        start(j + _RING)
    pltpu.sync_copy(rows_v, out_hbm.at[pl.ds(base, rows_per_worker)])

  return k


def kernel(feat, index):
  b, h, w, c = feat.shape
  n = index.shape[1]
  npad = n
  while (b * npad) % (8 * _NW) != 0:
    npad += 1
  rows_per_worker = b * npad // _NW
  idx_pad = jnp.pad(index, ((0, 0), (0, npad - n))).reshape(b * npad)
  n_full = w // 128
  batch_of = jnp.arange(b * npad, dtype=jnp.int32) // npad
  hh = idx_pad // w
  cw = idx_pad - hh * w
  rb = ((batch_of * h + hh) * c).astype(jnp.int32)
  wt = cw // 128
  lane = (cw - wt * 128).astype(jnp.int32)
  lane_b = jnp.broadcast_to(
      lane[:, None], (lane.shape[0], _L)).reshape(-1)
  tcw = jnp.transpose(feat, (0, 1, 3, 2)).reshape(b * h * c, w)
  k = _gather_kernel(h, w, c, rows_per_worker)
  out = k(tcw, rb, wt.astype(jnp.int32), lane_b)
  return out.reshape(b, npad, c)[:, :n, :]
